# Initial kernel scaffold; baseline (speedup 1.0000x reference)
import jax
import jax.numpy as jnp
from jax.experimental import pallas as pl


def kernel(x, table):
    raise NotImplementedError("write your pallas kernel here")


# speedup vs baseline: 1.0232x; 1.0232x over previous
import functools

import jax
import jax.numpy as jnp
from jax import lax
from jax.experimental import pallas as pl
from jax.experimental.pallas import tpu as pltpu
from jax.experimental.pallas import tpu_sc as plsc

_NC = 2
_NS = 16
_NW = _NC * _NS


@functools.lru_cache(maxsize=None)
def _emb_lookup(B, V, D, chunk):
    b_per_w = B // _NW
    n_chunks = b_per_w // chunk
    mesh = plsc.VectorSubcoreMesh(core_axis_name="c", subcore_axis_name="s")

    @functools.partial(
        pl.kernel,
        mesh=mesh,
        out_type=jax.ShapeDtypeStruct((B, D), jnp.float32),
        compiler_params=pltpu.CompilerParams(use_tc_tiling_on_sc=False),
        scratch_types=[
            pltpu.VMEM((b_per_w,), jnp.int32),
            pltpu.VMEM((chunk, D), jnp.float32),
            pltpu.SemaphoreType.DMA,
        ],
    )
    def emb(idx_hbm, table_hbm, out_hbm, idx_v, rows_v, gsem):
        wid = lax.axis_index("s") * _NC + lax.axis_index("c")
        base = wid * b_per_w
        pltpu.sync_copy(idx_hbm.at[pl.ds(base, b_per_w)], idx_v)

        def body(c, carry):
            off = pl.multiple_of(c * chunk, chunk)
            pltpu.async_copy(
                table_hbm.at[idx_v.at[pl.ds(off, chunk)]], rows_v, gsem
            ).wait()
            pltpu.sync_copy(rows_v, out_hbm.at[pl.ds(base + off, chunk)])
            return carry

        lax.fori_loop(0, n_chunks, body, 0)

    return emb


def kernel(x, table):
    B0, S = x.shape
    V, D = table.shape
    B = B0 * S
    idx = x.reshape(B).astype(jnp.int32)
    out = _emb_lookup(B, V, D, 128)(idx, table)
    return out.reshape(B0, S, D)


# speedup vs baseline: 1.1127x; 1.0875x over previous
import functools

import jax
import jax.numpy as jnp
from jax import lax
from jax.experimental import pallas as pl
from jax.experimental.pallas import tpu as pltpu
from jax.experimental.pallas import tpu_sc as plsc

_NC = 2
_NS = 16
_NW = _NC * _NS


@functools.lru_cache(maxsize=None)
def _emb_lookup(B, V, D, chunk, K):
    b_per_w = B // _NW
    n_chunks = b_per_w // chunk
    R = n_chunks // K
    assert n_chunks % K == 0 and R % 2 == 0 and b_per_w % chunk == 0
    mesh = plsc.VectorSubcoreMesh(core_axis_name="c", subcore_axis_name="s")

    @functools.partial(
        pl.kernel,
        mesh=mesh,
        out_type=jax.ShapeDtypeStruct((B, D), jnp.float32),
        compiler_params=pltpu.CompilerParams(use_tc_tiling_on_sc=False),
        scratch_types=[
            pltpu.VMEM((b_per_w,), jnp.int32),
            pltpu.VMEM((K, chunk, D), jnp.float32),
            pltpu.VMEM((K, chunk, D), jnp.float32),
            pltpu.SemaphoreType.DMA,
            pltpu.SemaphoreType.DMA,
            pltpu.SemaphoreType.DMA,
            pltpu.SemaphoreType.DMA,
        ],
    )
    def emb(idx_hbm, table_hbm, out_hbm, idx_v, rows_a, rows_b,
            gsem_a, gsem_b, osem_a, osem_b):
        wid = lax.axis_index("s") * _NC + lax.axis_index("c")
        base = wid * b_per_w
        pltpu.sync_copy(idx_hbm.at[pl.ds(base, b_per_w)], idx_v)

        def gather(r, bank, b, sem):
            off = pl.multiple_of((r * K + b) * chunk, chunk)
            return pltpu.make_async_copy(
                table_hbm.at[idx_v.at[pl.ds(off, chunk)]], bank.at[b], sem)

        def out_copy(r, bank, b, sem):
            off = pl.multiple_of((r * K + b) * chunk, chunk)
            return pltpu.make_async_copy(
                bank.at[b], out_hbm.at[pl.ds(base + off, chunk)], sem)

        for b in range(K):
            gather(0, rows_a, b, gsem_a).start()

        def body(rp, carry):
            r0 = 2 * rp
            r1 = r0 + 1

            for b in range(K):
                @pl.when(r0 > 0)
                def _(b=b):
                    out_copy(r1 - 2, rows_b, b, osem_b).wait()
                gather(r1, rows_b, b, gsem_b).start()
            for b in range(K):
                gather(r0, rows_a, b, gsem_a).wait()
                out_copy(r0, rows_a, b, osem_a).start()

            for b in range(K):
                @pl.when(r1 + 1 < R)
                def _(b=b):
                    out_copy(r0, rows_a, b, osem_a).wait()
                    gather(r1 + 1, rows_a, b, gsem_a).start()
            for b in range(K):
                gather(r1, rows_b, b, gsem_b).wait()
                out_copy(r1, rows_b, b, osem_b).start()
            return carry

        lax.fori_loop(0, R // 2, body, 0)

        for b in range(K):
            out_copy(R - 2, rows_a, b, osem_a).wait()
            out_copy(R - 1, rows_b, b, osem_b).wait()

    return emb


def kernel(x, table):
    B0, S = x.shape
    V, D = table.shape
    B = B0 * S
    idx = x.reshape(B).astype(jnp.int32)
    out = _emb_lookup(B, V, D, 256, 5)(idx, table)
    return out.reshape(B0, S, D)


# speedup vs baseline: 1.2588x; 1.1313x over previous
import functools

import jax
import jax.numpy as jnp
from jax import lax
from jax.experimental import pallas as pl
from jax.experimental.pallas import tpu as pltpu
from jax.experimental.pallas import tpu_sc as plsc

_NC = 2
_NS = 16
_NW = _NC * _NS
_MESH = plsc.VectorSubcoreMesh(core_axis_name="c", subcore_axis_name="s")


@functools.lru_cache(maxsize=None)
def _stage_a(V, D, B0, S):
    n_tc = V // 128
    tail = V - n_tc * 128
    per_w = n_tc // _NW
    extra = n_tc - per_w * _NW
    bw = B0 // _NW

    @functools.partial(
        pl.kernel,
        mesh=_MESH,
        out_type=(
            jax.ShapeDtypeStruct((V * D // 128, 128), jnp.float32),
            jax.ShapeDtypeStruct((B0 * S,), jnp.int32),
        ),
        compiler_params=pltpu.CompilerParams(needs_layout_passes=False),
        scratch_types=[
            pltpu.VMEM((S, bw), jnp.int32),
            pltpu.VMEM((2, D, 128), jnp.float32),
            pltpu.VMEM((2, 32, 128), jnp.float32),
            pltpu.VMEM((64, D), jnp.float32),
            pltpu.SemaphoreType.DMA,
            pltpu.SemaphoreType.DMA,
            pltpu.SemaphoreType.DMA,
            pltpu.SemaphoreType.DMA,
            pltpu.SemaphoreType.DMA,
        ],
    )
    def a(tt_hbm, xt_hbm, ttail_hbm, tlin_hbm, idx_hbm, xbuf, sbuf, dbuf, tbuf,
          gsem0, gsem1, wsem0, wsem1, xsem):
        wid = lax.axis_index("s") * _NC + lax.axis_index("c")

        pltpu.sync_copy(xt_hbm.at[:, pl.ds(bw * wid, bw)], xbuf)
        for s in range(S):
            pltpu.make_async_copy(
                xbuf.at[s], idx_hbm.at[pl.ds(s * B0 + bw * wid, bw)], xsem
            ).start()

        iota = lax.iota(jnp.int32, 16)
        idx_d0 = iota
        idx_d1 = iota + 16

        def tcol(t):
            return wid + t * _NW

        def fetch(t, buf, sem):
            return pltpu.make_async_copy(
                tt_hbm.at[:, pl.ds(tcol(t) * 128, 128)], buf, sem)

        def wout(t, buf, sem):
            return pltpu.make_async_copy(
                buf, tlin_hbm.at[pl.ds(tcol(t) * 32, 32), :], sem)

        def transpose_block(src, dst):
            def body(jp, carry):
                for m in range(8):
                    idx_d = idx_d1 if (m % 2) else idx_d0
                    idx_l = jnp.full((16,), 4 * jp + (m // 2), jnp.int32)
                    dst[jp, pl.ds(16 * m, 16)] = plsc.load_gather(
                        src, [idx_d, idx_l])
                return carry
            lax.fori_loop(0, 32, body, 0)

        n_t = per_w + 1
        n_mine = per_w + jnp.where(wid < extra, 1, 0)
        fetch(0, sbuf.at[0], gsem0).start()

        def step(t, carry):
            for p in range(2):
                tt_i = 2 * t + p
                sem = gsem1 if p else gsem0
                wsem = wsem1 if p else wsem0
                nsem = gsem0 if p else gsem1

                @pl.when(tt_i < n_mine)
                def _(tt_i=tt_i, p=p, sem=sem, wsem=wsem, nsem=nsem):
                    @pl.when(tt_i + 1 < n_mine)
                    def _():
                        fetch(tt_i + 1, sbuf.at[1 - p], nsem).start()
                    fetch(tt_i, sbuf.at[p], sem).wait()
                    @pl.when(tt_i >= 2)
                    def _():
                        wout(tt_i - 2, dbuf.at[p], wsem).wait()
                    transpose_block(sbuf.at[p], dbuf.at[p])
                    wout(tt_i, dbuf.at[p], wsem).start()
            return carry

        lax.fori_loop(0, (n_t + 1) // 2, step, 0)

        for p in range(2):
            wsem = wsem1 if p else wsem0
            pltpu.make_async_copy(
                dbuf.at[p], tlin_hbm.at[pl.ds(0, 32), :], wsem).wait()

        @pl.when(wid == _NW - 1)
        def _():
            pltpu.sync_copy(ttail_hbm, tbuf)

            def body(jp, carry):
                for m in range(8):
                    idx_d = idx_d1 if (m % 2) else idx_d0
                    idx_v = jnp.full((16,), 4 * jp + (m // 2), jnp.int32)
                    dbuf[0, jp, pl.ds(16 * m, 16)] = plsc.load_gather(
                        tbuf, [idx_v, idx_d])
                return carry
            lax.fori_loop(0, tail // 4, body, 0)
            pltpu.sync_copy(dbuf.at[0, pl.ds(0, tail // 4)],
                            tlin_hbm.at[pl.ds(n_tc * 32, tail // 4), :])

        for s in range(S):
            pltpu.make_async_copy(
                xbuf.at[0], idx_hbm.at[pl.ds(0, bw)], xsem).wait()

    return a


@functools.lru_cache(maxsize=None)
def _stage_b(B, V, D, chunk, K):
    b_per_w = B // _NW
    n_chunks = b_per_w // chunk
    R = n_chunks // K
    assert n_chunks % K == 0 and R % 2 == 0

    @functools.partial(
        pl.kernel,
        mesh=_MESH,
        out_type=jax.ShapeDtypeStruct((B, D), jnp.float32),
        compiler_params=pltpu.CompilerParams(use_tc_tiling_on_sc=False),
        scratch_types=[
            pltpu.VMEM((b_per_w,), jnp.int32),
            pltpu.VMEM((K, chunk, D), jnp.float32),
            pltpu.VMEM((K, chunk, D), jnp.float32),
            pltpu.SemaphoreType.DMA,
            pltpu.SemaphoreType.DMA,
            pltpu.SemaphoreType.DMA,
            pltpu.SemaphoreType.DMA,
        ],
    )
    def b(idx_hbm, table_hbm, out_hbm, idx_v, rows_a, rows_b,
          gsem_a, gsem_b, osem_a, osem_b):
        wid = lax.axis_index("s") * _NC + lax.axis_index("c")
        base = wid * b_per_w
        pltpu.sync_copy(idx_hbm.at[pl.ds(base, b_per_w)], idx_v)

        def gather(r, bank, bb, sem):
            off = pl.multiple_of((r * K + bb) * chunk, chunk)
            return pltpu.make_async_copy(
                table_hbm.at[idx_v.at[pl.ds(off, chunk)]], bank.at[bb], sem)

        def out_copy(r, bank, bb, sem):
            off = pl.multiple_of((r * K + bb) * chunk, chunk)
            return pltpu.make_async_copy(
                bank.at[bb], out_hbm.at[pl.ds(base + off, chunk)], sem)

        for bb in range(K):
            gather(0, rows_a, bb, gsem_a).start()

        def body(rp, carry):
            r0 = 2 * rp
            r1 = r0 + 1
            for bb in range(K):
                @pl.when(r0 > 0)
                def _(bb=bb):
                    out_copy(r1 - 2, rows_b, bb, osem_b).wait()
                gather(r1, rows_b, bb, gsem_b).start()
            for bb in range(K):
                gather(r0, rows_a, bb, gsem_a).wait()
                out_copy(r0, rows_a, bb, osem_a).start()
            for bb in range(K):
                @pl.when(r1 + 1 < R)
                def _(bb=bb):
                    out_copy(r0, rows_a, bb, osem_a).wait()
                    gather(r1 + 1, rows_a, bb, gsem_a).start()
            for bb in range(K):
                gather(r1, rows_b, bb, gsem_b).wait()
                out_copy(r1, rows_b, bb, osem_b).start()
            return carry

        lax.fori_loop(0, R // 2, body, 0)
        for bb in range(K):
            out_copy(R - 2, rows_a, bb, osem_a).wait()
            out_copy(R - 1, rows_b, bb, osem_b).wait()

    return b


@functools.lru_cache(maxsize=None)
def _stage_c(B0, S, D):
    n_blocks = S * (B0 // 128)
    per_w = n_blocks // _NW

    @functools.partial(
        pl.kernel,
        mesh=_MESH,
        out_type=jax.ShapeDtypeStruct((S, D, B0), jnp.float32),
        compiler_params=pltpu.CompilerParams(needs_layout_passes=False),
        scratch_types=[
            pltpu.VMEM((2, 32, 128), jnp.float32),
            pltpu.VMEM((2, D, 128), jnp.float32),
            pltpu.SemaphoreType.DMA,
            pltpu.SemaphoreType.DMA,
            pltpu.SemaphoreType.DMA,
            pltpu.SemaphoreType.DMA,
        ],
    )
    def c(gv_hbm, out_hbm, sbuf, dbuf, gsem0, gsem1, wsem0, wsem1):
        wid = lax.axis_index("s") * _NC + lax.axis_index("c")
        beta0 = wid * per_w

        iota = lax.iota(jnp.int32, 16)
        idx_j_base = iota // 4
        idx_l_base = (iota % 4) * 32

        def fetch(t, buf, sem):
            beta = beta0 + t
            s = beta // 128
            tc = beta % 128
            return pltpu.make_async_copy(
                gv_hbm.at[pl.ds(4096 * s + 32 * tc, 32), :], buf, sem)

        def wout(t, buf, sem):
            beta = beta0 + t
            s = beta // 128
            tc = beta % 128
            return pltpu.make_async_copy(
                buf, out_hbm.at[s, :, pl.ds(128 * tc, 128)], sem)

        def transpose_block(src, dst):
            def body(d, carry):
                idx_l = idx_l_base + d
                for m in range(8):
                    dst[d, pl.ds(16 * m, 16)] = plsc.load_gather(
                        src, [idx_j_base + 4 * m, idx_l])
                return carry
            lax.fori_loop(0, D, body, 0)

        fetch(0, sbuf.at[0], gsem0).start()

        def step(t, carry):
            for p in range(2):
                tt_i = 2 * t + p
                sem = gsem1 if p else gsem0
                wsem = wsem1 if p else wsem0
                nsem = gsem0 if p else gsem1
                @pl.when(tt_i + 1 < per_w)
                def _(tt_i=tt_i, p=p, nsem=nsem):
                    fetch(tt_i + 1, sbuf.at[1 - p], nsem).start()
                fetch(tt_i, sbuf.at[p], sem).wait()
                @pl.when(tt_i >= 2)
                def _(tt_i=tt_i, p=p, wsem=wsem):
                    wout(tt_i - 2, dbuf.at[p], wsem).wait()
                transpose_block(sbuf.at[p], dbuf.at[p])
                wout(tt_i, dbuf.at[p], wsem).start()
            return carry

        lax.fori_loop(0, per_w // 2, step, 0)
        for p in range(2):
            wsem = wsem1 if p else wsem0
            pltpu.make_async_copy(
                dbuf.at[p], out_hbm.at[0, :, pl.ds(0, 128)], wsem).wait()

    return c


def kernel(x, table):
    B0, S = x.shape
    V, D = table.shape
    B = B0 * S
    tlin, idxr = _stage_a(V, D, B0, S)(table.T, x.T, table[V - (V % 128):])
    tfl2 = tlin.reshape(V, D)
    g = _stage_b(B, V, D, 256, 5)(idxr, tfl2)
    gv = g.reshape(B * D // 128, 128)
    out_t = _stage_c(B0, S, D)(gv)
    return out_t.transpose(2, 0, 1)


# speedup vs baseline: 1.5252x; 1.2116x over previous
import functools

import jax
import jax.numpy as jnp
from jax import lax
from jax.experimental import pallas as pl
from jax.experimental.pallas import tpu as pltpu
from jax.experimental.pallas import tpu_sc as plsc

_NC = 2
_NS = 16
_NW = _NC * _NS
_MESH = plsc.VectorSubcoreMesh(core_axis_name="c", subcore_axis_name="s")


@functools.lru_cache(maxsize=None)
def _stage_a(V, D, B0, S):
    n_tc = V // 128
    tail = V - n_tc * 128
    per_w = n_tc // _NW
    extra = n_tc - per_w * _NW
    bw = B0 // _NW

    @functools.partial(
        pl.kernel,
        mesh=_MESH,
        out_type=(
            jax.ShapeDtypeStruct((V * D,), jnp.float32),
            jax.ShapeDtypeStruct((B0 * S,), jnp.int32),
        ),
        compiler_params=pltpu.CompilerParams(needs_layout_passes=False),
        scratch_types=[
            pltpu.VMEM((S, bw), jnp.int32),
            pltpu.VMEM((D, 128), jnp.float32),
            pltpu.VMEM((D, 128), jnp.float32),
            pltpu.VMEM((4096,), jnp.float32),
            pltpu.VMEM((4096,), jnp.float32),
            pltpu.VMEM((64, D), jnp.float32),
            pltpu.SemaphoreType.DMA,
            pltpu.SemaphoreType.DMA,
            pltpu.SemaphoreType.DMA,
            pltpu.SemaphoreType.DMA,
            pltpu.SemaphoreType.DMA,
        ],
    )
    def a(tt_hbm, xt_hbm, ttail_hbm, tlin_hbm, idx_hbm, xbuf, sbuf0, sbuf1,
          dbuf0, dbuf1, tbuf, gsem0, gsem1, wsem0, wsem1, xsem):
        wid = lax.axis_index("s") * _NC + lax.axis_index("c")

        pltpu.sync_copy(xt_hbm.at[:, pl.ds(bw * wid, bw)], xbuf)
        for s in range(S):
            pltpu.make_async_copy(
                xbuf.at[s], idx_hbm.at[pl.ds(s * B0 + bw * wid, bw)], xsem
            ).start()

        iota = lax.iota(jnp.int32, 16)
        pbase = (iota // 4) * 128 + (iota % 4) * 32

        def tcol(t):
            return wid + t * _NW

        def fetch(t, buf, sem):
            return pltpu.make_async_copy(
                tt_hbm.at[:, pl.ds(tcol(t) * 128, 128)], buf, sem)

        def wout(t, buf, sem):
            return pltpu.make_async_copy(
                buf, tlin_hbm.at[pl.ds(tcol(t) * 4096, 4096)], sem)

        def transpose_block(src, dst):
            for d in range(D):
                for h in range(8):
                    vec = src[d, pl.ds(16 * h, 16)]
                    plsc.store_scatter(dst, [pbase + (512 * h + d)], vec)

        n_t = per_w + 1
        n_mine = per_w + jnp.where(wid < extra, 1, 0)
        fetch(0, sbuf0, gsem0).start()

        def step(t, carry):
            for p in range(2):
                tt_i = 2 * t + p
                sem = gsem1 if p else gsem0
                wsem = wsem1 if p else wsem0
                nsem = gsem0 if p else gsem1
                sb = sbuf1 if p else sbuf0
                nsb = sbuf0 if p else sbuf1
                db = dbuf1 if p else dbuf0

                @pl.when(tt_i < n_mine)
                def _(tt_i=tt_i, sem=sem, wsem=wsem, nsem=nsem,
                      sb=sb, nsb=nsb, db=db):
                    @pl.when(tt_i + 1 < n_mine)
                    def _():
                        fetch(tt_i + 1, nsb, nsem).start()
                    fetch(tt_i, sb, sem).wait()
                    @pl.when(tt_i >= 2)
                    def _():
                        wout(tt_i - 2, db, wsem).wait()
                    transpose_block(sb, db)
                    wout(tt_i, db, wsem).start()
            return carry

        lax.fori_loop(0, (n_t + 1) // 2, step, 0)

        for p in range(2):
            wsem = wsem1 if p else wsem0
            db = dbuf1 if p else dbuf0
            pltpu.make_async_copy(
                db, tlin_hbm.at[pl.ds(0, 4096)], wsem).wait()

        @pl.when(wid == _NW - 1)
        def _():
            pltpu.sync_copy(ttail_hbm, tbuf)
            for vp in range(tail):
                for h2 in range(D // 16):
                    vec = tbuf[vp, pl.ds(16 * h2, 16)]
                    off = (vp // 4) * 128 + 32 * (vp % 4) + 16 * h2
                    plsc.store_scatter(dbuf0, [iota + off], vec)
            pltpu.sync_copy(dbuf0.at[pl.ds(0, tail * D)],
                            tlin_hbm.at[pl.ds(n_tc * 4096, tail * D)])

        for s in range(S):
            pltpu.make_async_copy(
                xbuf.at[0], idx_hbm.at[pl.ds(0, bw)], xsem).wait()

    return a


@functools.lru_cache(maxsize=None)
def _stage_b(B, V, D, chunk, K):
    b_per_w = B // _NW
    n_chunks = b_per_w // chunk
    R = n_chunks // K
    assert n_chunks % K == 0 and R % 2 == 0

    @functools.partial(
        pl.kernel,
        mesh=_MESH,
        out_type=jax.ShapeDtypeStruct((B, D), jnp.float32),
        compiler_params=pltpu.CompilerParams(use_tc_tiling_on_sc=False),
        scratch_types=[
            pltpu.VMEM((b_per_w,), jnp.int32),
            pltpu.VMEM((K, chunk, D), jnp.float32),
            pltpu.VMEM((K, chunk, D), jnp.float32),
            pltpu.SemaphoreType.DMA,
            pltpu.SemaphoreType.DMA,
            pltpu.SemaphoreType.DMA,
            pltpu.SemaphoreType.DMA,
        ],
    )
    def b(idx_hbm, table_hbm, out_hbm, idx_v, rows_a, rows_b,
          gsem_a, gsem_b, osem_a, osem_b):
        wid = lax.axis_index("s") * _NC + lax.axis_index("c")
        base = wid * b_per_w
        pltpu.sync_copy(idx_hbm.at[pl.ds(base, b_per_w)], idx_v)

        def gather(r, bank, bb, sem):
            off = pl.multiple_of((r * K + bb) * chunk, chunk)
            return pltpu.make_async_copy(
                table_hbm.at[idx_v.at[pl.ds(off, chunk)]], bank.at[bb], sem)

        def out_copy(r, bank, bb, sem):
            off = pl.multiple_of((r * K + bb) * chunk, chunk)
            return pltpu.make_async_copy(
                bank.at[bb], out_hbm.at[pl.ds(base + off, chunk)], sem)

        for bb in range(K):
            gather(0, rows_a, bb, gsem_a).start()

        def body(rp, carry):
            r0 = 2 * rp
            r1 = r0 + 1
            for bb in range(K):
                @pl.when(r0 > 0)
                def _(bb=bb):
                    out_copy(r1 - 2, rows_b, bb, osem_b).wait()
                gather(r1, rows_b, bb, gsem_b).start()
            for bb in range(K):
                gather(r0, rows_a, bb, gsem_a).wait()
                out_copy(r0, rows_a, bb, osem_a).start()
            for bb in range(K):
                @pl.when(r1 + 1 < R)
                def _(bb=bb):
                    out_copy(r0, rows_a, bb, osem_a).wait()
                    gather(r1 + 1, rows_a, bb, gsem_a).start()
            for bb in range(K):
                gather(r1, rows_b, bb, gsem_b).wait()
                out_copy(r1, rows_b, bb, osem_b).start()
            return carry

        lax.fori_loop(0, R // 2, body, 0)
        for bb in range(K):
            out_copy(R - 2, rows_a, bb, osem_a).wait()
            out_copy(R - 1, rows_b, bb, osem_b).wait()

    return b


@functools.lru_cache(maxsize=None)
def _stage_c(B0, S, D):
    n_blocks = S * (B0 // 128)
    per_w = n_blocks // _NW

    @functools.partial(
        pl.kernel,
        mesh=_MESH,
        out_type=jax.ShapeDtypeStruct((S, D, B0), jnp.float32),
        compiler_params=pltpu.CompilerParams(needs_layout_passes=False),
        scratch_types=[
            pltpu.VMEM((32, 128), jnp.float32),
            pltpu.VMEM((32, 128), jnp.float32),
            pltpu.VMEM((D, 128), jnp.float32),
            pltpu.VMEM((D, 128), jnp.float32),
            pltpu.SemaphoreType.DMA,
            pltpu.SemaphoreType.DMA,
            pltpu.SemaphoreType.DMA,
            pltpu.SemaphoreType.DMA,
        ],
    )
    def c(gv_hbm, out_hbm, sbuf0, sbuf1, dbuf0, dbuf1,
          gsem0, gsem1, wsem0, wsem1):
        wid = lax.axis_index("s") * _NC + lax.axis_index("c")
        beta0 = wid * per_w

        iota = lax.iota(jnp.int32, 16)
        row0 = iota
        row1 = iota + 16

        def fetch(t, buf, sem):
            beta = beta0 + t
            s = beta // 128
            tc = beta % 128
            return pltpu.make_async_copy(
                gv_hbm.at[pl.ds(4096 * s + 32 * tc, 32), :], buf, sem)

        def wout(t, buf, sem):
            beta = beta0 + t
            s = beta // 128
            tc = beta % 128
            return pltpu.make_async_copy(
                buf, out_hbm.at[s, :, pl.ds(128 * tc, 128)], sem)

        def transpose_block(src, dst):
            for jp in range(32):
                for h in range(8):
                    vec = src[jp, pl.ds(16 * h, 16)]
                    idx_r = row1 if (h % 2) else row0
                    idx_l = jnp.full((16,), 4 * jp + h // 2, jnp.int32)
                    plsc.store_scatter(dst, [idx_r, idx_l], vec)

        fetch(0, sbuf0, gsem0).start()

        def step(t, carry):
            for p in range(2):
                tt_i = 2 * t + p
                sem = gsem1 if p else gsem0
                wsem = wsem1 if p else wsem0
                nsem = gsem0 if p else gsem1
                sb = sbuf1 if p else sbuf0
                nsb = sbuf0 if p else sbuf1
                db = dbuf1 if p else dbuf0
                @pl.when(tt_i + 1 < per_w)
                def _(tt_i=tt_i, nsb=nsb, nsem=nsem):
                    fetch(tt_i + 1, nsb, nsem).start()
                fetch(tt_i, sb, sem).wait()
                @pl.when(tt_i >= 2)
                def _(tt_i=tt_i, db=db, wsem=wsem):
                    wout(tt_i - 2, db, wsem).wait()
                transpose_block(sb, db)
                wout(tt_i, db, wsem).start()
            return carry

        lax.fori_loop(0, per_w // 2, step, 0)
        for p in range(2):
            wsem = wsem1 if p else wsem0
            db = dbuf1 if p else dbuf0
            pltpu.make_async_copy(
                db, out_hbm.at[0, :, pl.ds(0, 128)], wsem).wait()

    return c


def kernel(x, table):
    B0, S = x.shape
    V, D = table.shape
    B = B0 * S
    tlin, idxr = _stage_a(V, D, B0, S)(table.T, x.T, table[V - (V % 128):])
    tfl2 = tlin.reshape(V, D)
    g = _stage_b(B, V, D, 256, 5)(idxr, tfl2)
    gv = g.reshape(B * D // 128, 128)
    out_t = _stage_c(B0, S, D)(gv)
    return out_t.transpose(2, 0, 1)
